# Initial kernel scaffold; baseline (speedup 1.0000x reference)
import jax
import jax.numpy as jnp
from jax.experimental import pallas as pl


def kernel(feature1, coord1, feature2, coord2, W1, b1, g1, be1, W2, b2, g2, be2):
    raise NotImplementedError("write your pallas kernel here")


# speedup vs baseline: 19.7121x; 19.7121x over previous
import functools

import jax
import jax.numpy as jnp
from jax.experimental import pallas as pl


def _stage1(c1_ref, c2_ref, f1_ref, f2_ref, w1a_ref, w1b_ref,
            x1_ref, st_ref, *, blk_n, m):
    c1 = c1_ref[0]
    c2 = c2_ref[0]
    cross = jax.lax.dot_general(
        c1, c2, (((1,), (0,)), ((), ())),
        precision=jax.lax.Precision.DEFAULT,
        preferred_element_type=jnp.float32)
    s1 = c1[:, 0:1] ** 2 + c1[:, 1:2] ** 2 + c1[:, 2:3] ** 2
    s2 = c2[0:1, :] ** 2 + c2[1:2, :] ** 2 + c2[2:3, :] ** 2
    d = (-2.0 * cross + s1) + s2

    iota = jax.lax.broadcasted_iota(jnp.int32, (blk_n, m), 1)
    big = jnp.float32(1e30)
    wrow = jnp.zeros((blk_n, m), jnp.float32)
    recips = []
    onehots = []
    for _ in range(3):
        mj = jnp.min(d, axis=1, keepdims=True)
        ij = jnp.min(jnp.where(d == mj, iota, m), axis=1,
                     keepdims=True)
        hit = (iota == ij)
        recips.append(1.0 / (mj + 1e-8))
        onehots.append(hit)
        d = jnp.where(hit, big, d)
    norm = recips[0] + recips[1] + recips[2]
    for r, h in zip(recips, onehots):
        wrow = wrow + jnp.where(h, r / norm, 0.0)

    f2 = f2_ref[0]
    interp = jax.lax.dot_general(
        wrow, f2, (((1,), (1,)), ((), ())),
        precision=jax.lax.Precision.HIGHEST,
        preferred_element_type=jnp.float32)

    f1 = f1_ref[0]
    xa = jax.lax.dot_general(
        f1, w1a_ref[...], (((0,), (0,)), ((), ())),
        preferred_element_type=jnp.float32)
    xb = jnp.dot(interp, w1b_ref[...],
                 preferred_element_type=jnp.float32)
    x = xa + xb
    x1_ref[0] = x

    s = jnp.sum(x, axis=0)
    sq = jnp.sum(x * x, axis=0)
    st_ref[0, 0] = jnp.concatenate([s[None, :], sq[None, :]], axis=0)


def _bn_from_partials(st_ref, n_total):
    st = jnp.sum(st_ref[...], axis=(0, 1))
    mean = st[0:1, :] / n_total
    var = st[1:2, :] / n_total - mean * mean
    rstd = jax.lax.rsqrt(var + 1e-5)
    return mean, rstd


def _stage2(x1_ref, st_ref, g1_ref, be1_ref, w2t_ref, x2_ref, st2_ref,
            *, n_total):
    x = x1_ref[0]
    mean, rstd = _bn_from_partials(st_ref, n_total)
    h = (x - mean) * (rstd * g1_ref[...]) + be1_ref[...]
    h = jnp.maximum(h, 0.0)
    y = jnp.dot(h, w2t_ref[...], preferred_element_type=jnp.float32)
    x2_ref[0] = y
    s = jnp.sum(y, axis=0)
    sq = jnp.sum(y * y, axis=0)
    st2_ref[0, 0] = jnp.concatenate([s[None, :], sq[None, :]], axis=0)


def _stage3(x2_ref, st2_ref, g2_ref, be2_ref, out_ref, *, n_total):
    x = x2_ref[0]
    mean, rstd = _bn_from_partials(st2_ref, n_total)
    h = (x - mean) * (rstd * g2_ref[...]) + be2_ref[...]
    h = jnp.maximum(h, 0.0)
    out_ref[0] = h.T


def kernel(feature1, coord1, feature2, coord2, W1, b1, g1, be1, W2, b2, g2, be2):
    B, C1, N = feature1.shape
    _, C2, M = feature2.shape
    F1 = W1.shape[0]
    F2 = W2.shape[0]
    BLK = 512
    NB = N // BLK
    n_total = float(B * N)

    c1t = coord1.transpose(0, 2, 1)
    w1a = W1[:, :C1].T
    w1b = W1[:, C1:].T
    w2t = W2.T
    g1r, be1r = g1.reshape(1, F1), be1.reshape(1, F1)
    g2r, be2r = g2.reshape(1, F2), be2.reshape(1, F2)

    x1, st1 = pl.pallas_call(
        functools.partial(_stage1, blk_n=BLK, m=M),
        grid=(B, NB),
        in_specs=[
            pl.BlockSpec((1, BLK, 3), lambda b, nb: (b, nb, 0)),
            pl.BlockSpec((1, 3, M), lambda b, nb: (b, 0, 0)),
            pl.BlockSpec((1, C1, BLK), lambda b, nb: (b, 0, nb)),
            pl.BlockSpec((1, C2, M), lambda b, nb: (b, 0, 0)),
            pl.BlockSpec((C1, F1), lambda b, nb: (0, 0)),
            pl.BlockSpec((C2, F1), lambda b, nb: (0, 0)),
        ],
        out_specs=[
            pl.BlockSpec((1, BLK, F1), lambda b, nb: (b, nb, 0)),
            pl.BlockSpec((1, 1, 2, F1), lambda b, nb: (b, nb, 0, 0)),
        ],
        out_shape=[
            jax.ShapeDtypeStruct((B, N, F1), jnp.float32),
            jax.ShapeDtypeStruct((B, NB, 2, F1), jnp.float32),
        ],
    )(c1t, coord2, feature1, feature2, w1a, w1b)

    x2, st2 = pl.pallas_call(
        functools.partial(_stage2, n_total=n_total),
        grid=(B, NB),
        in_specs=[
            pl.BlockSpec((1, BLK, F1), lambda b, nb: (b, nb, 0)),
            pl.BlockSpec((B, NB, 2, F1), lambda b, nb: (0, 0, 0, 0)),
            pl.BlockSpec((1, F1), lambda b, nb: (0, 0)),
            pl.BlockSpec((1, F1), lambda b, nb: (0, 0)),
            pl.BlockSpec((F1, F2), lambda b, nb: (0, 0)),
        ],
        out_specs=[
            pl.BlockSpec((1, BLK, F2), lambda b, nb: (b, nb, 0)),
            pl.BlockSpec((1, 1, 2, F2), lambda b, nb: (b, nb, 0, 0)),
        ],
        out_shape=[
            jax.ShapeDtypeStruct((B, N, F2), jnp.float32),
            jax.ShapeDtypeStruct((B, NB, 2, F2), jnp.float32),
        ],
    )(x1, st1, g1r, be1r, w2t)

    out = pl.pallas_call(
        functools.partial(_stage3, n_total=n_total),
        grid=(B, NB),
        in_specs=[
            pl.BlockSpec((1, BLK, F2), lambda b, nb: (b, nb, 0)),
            pl.BlockSpec((B, NB, 2, F2), lambda b, nb: (0, 0, 0, 0)),
            pl.BlockSpec((1, F2), lambda b, nb: (0, 0)),
            pl.BlockSpec((1, F2), lambda b, nb: (0, 0)),
        ],
        out_specs=pl.BlockSpec((1, F2, BLK), lambda b, nb: (b, 0, nb)),
        out_shape=jax.ShapeDtypeStruct((B, F2, N), jnp.float32),
    )(x2, st2, g2r, be2r)

    return out


# speedup vs baseline: 29.0761x; 1.4750x over previous
import functools

import jax
import jax.numpy as jnp
from jax.experimental import pallas as pl


def _stage1(c1_ref, c2_ref, f1_ref, f2hi_ref, w1a_ref, w1b_ref,
            x1_ref, st_ref, *, blk_n, m):
    c1 = c1_ref[0]
    c2 = c2_ref[0]
    cross = jax.lax.dot_general(
        c1, c2, (((1,), (0,)), ((), ())),
        precision=jax.lax.Precision.DEFAULT,
        preferred_element_type=jnp.float32)
    s1 = c1[:, 0:1] ** 2 + c1[:, 1:2] ** 2 + c1[:, 2:3] ** 2
    s2 = c2[0:1, :] ** 2 + c2[1:2, :] ** 2 + c2[2:3, :] ** 2
    d = (-2.0 * cross + s1) + s2

    big = jnp.float32(1e30)
    recips = []
    masks = []
    counts = []
    for _ in range(3):
        mj = jnp.min(d, axis=1, keepdims=True)
        e = (d == mj)
        ef = e.astype(jnp.float32)
        counts.append(jnp.sum(ef, axis=1, keepdims=True))
        recips.append(1.0 / (mj + 1e-8))
        masks.append(ef)
        d = jnp.where(e, big, d)
    three = jnp.float32(3.0)
    take1 = jnp.minimum(counts[0], three)
    take2 = jnp.minimum(counts[1], three - take1)
    take3 = jnp.minimum(counts[2], three - take1 - take2)
    takes = [take1, take2, take3]
    norm = take1 * recips[0] + take2 * recips[1] + take3 * recips[2]

    wrow = jnp.zeros((blk_n, m), jnp.float32)
    for r, ef, c, t in zip(recips, masks, counts, takes):
        wrow = wrow + ef * (r * t / (c * norm))
    f2 = f2hi_ref[0]
    dims = (((1,), (1,)), ((), ()))
    interp = jax.lax.dot_general(wrow, f2, dims,
                                 preferred_element_type=jnp.float32)

    f1 = f1_ref[0]
    xa = jax.lax.dot_general(
        f1, w1a_ref[...], (((0,), (0,)), ((), ())),
        preferred_element_type=jnp.float32)
    xb = jnp.dot(interp, w1b_ref[...],
                 preferred_element_type=jnp.float32)
    x = xa + xb
    x1_ref[0] = x

    s = jnp.sum(x, axis=0)
    sq = jnp.sum(x * x, axis=0)
    st_ref[0, 0] = jnp.concatenate([s[None, :], sq[None, :]], axis=0)


def _bn_from_partials(st_ref, n_total):
    st = jnp.sum(st_ref[...], axis=(0, 1))
    mean = st[0:1, :] / n_total
    var = st[1:2, :] / n_total - mean * mean
    rstd = jax.lax.rsqrt(var + 1e-5)
    return mean, rstd


def _stage2(x1_ref, st_ref, g1_ref, be1_ref, w2t_ref, x2_ref, st2_ref,
            *, n_total):
    x = x1_ref[0]
    mean, rstd = _bn_from_partials(st_ref, n_total)
    h = (x - mean) * (rstd * g1_ref[...]) + be1_ref[...]
    h = jnp.maximum(h, 0.0)
    y = jnp.dot(h, w2t_ref[...], preferred_element_type=jnp.float32)
    x2_ref[0] = y
    s = jnp.sum(y, axis=0)
    sq = jnp.sum(y * y, axis=0)
    st2_ref[0, 0] = jnp.concatenate([s[None, :], sq[None, :]], axis=0)


def _stage3(x2_ref, st2_ref, g2_ref, be2_ref, out_ref, *, n_total):
    x = x2_ref[0]
    mean, rstd = _bn_from_partials(st2_ref, n_total)
    h = (x - mean) * (rstd * g2_ref[...]) + be2_ref[...]
    h = jnp.maximum(h, 0.0)
    out_ref[0] = h.T


def kernel(feature1, coord1, feature2, coord2, W1, b1, g1, be1, W2, b2, g2, be2):
    B, C1, N = feature1.shape
    _, C2, M = feature2.shape
    F1 = W1.shape[0]
    F2 = W2.shape[0]
    BLK = 512
    NB = N // BLK
    n_total = float(B * N)

    c1t = coord1.transpose(0, 2, 1)
    w1a = W1[:, :C1].T
    w1b = W1[:, C1:].T
    w2t = W2.T
    g1r, be1r = g1.reshape(1, F1), be1.reshape(1, F1)
    g2r, be2r = g2.reshape(1, F2), be2.reshape(1, F2)

    x1, st1 = pl.pallas_call(
        functools.partial(_stage1, blk_n=BLK, m=M),
        grid=(B, NB),
        in_specs=[
            pl.BlockSpec((1, BLK, 3), lambda b, nb: (b, nb, 0)),
            pl.BlockSpec((1, 3, M), lambda b, nb: (b, 0, 0)),
            pl.BlockSpec((1, C1, BLK), lambda b, nb: (b, 0, nb)),
            pl.BlockSpec((1, C2, M), lambda b, nb: (b, 0, 0)),
            pl.BlockSpec((C1, F1), lambda b, nb: (0, 0)),
            pl.BlockSpec((C2, F1), lambda b, nb: (0, 0)),
        ],
        out_specs=[
            pl.BlockSpec((1, BLK, F1), lambda b, nb: (b, nb, 0)),
            pl.BlockSpec((1, 1, 2, F1), lambda b, nb: (b, nb, 0, 0)),
        ],
        out_shape=[
            jax.ShapeDtypeStruct((B, N, F1), jnp.float32),
            jax.ShapeDtypeStruct((B, NB, 2, F1), jnp.float32),
        ],
    )(c1t, coord2, feature1, feature2, w1a, w1b)

    x2, st2 = pl.pallas_call(
        functools.partial(_stage2, n_total=n_total),
        grid=(B, NB),
        in_specs=[
            pl.BlockSpec((1, BLK, F1), lambda b, nb: (b, nb, 0)),
            pl.BlockSpec((B, NB, 2, F1), lambda b, nb: (0, 0, 0, 0)),
            pl.BlockSpec((1, F1), lambda b, nb: (0, 0)),
            pl.BlockSpec((1, F1), lambda b, nb: (0, 0)),
            pl.BlockSpec((F1, F2), lambda b, nb: (0, 0)),
        ],
        out_specs=[
            pl.BlockSpec((1, BLK, F2), lambda b, nb: (b, nb, 0)),
            pl.BlockSpec((1, 1, 2, F2), lambda b, nb: (b, nb, 0, 0)),
        ],
        out_shape=[
            jax.ShapeDtypeStruct((B, N, F2), jnp.float32),
            jax.ShapeDtypeStruct((B, NB, 2, F2), jnp.float32),
        ],
    )(x1, st1, g1r, be1r, w2t)

    out = pl.pallas_call(
        functools.partial(_stage3, n_total=n_total),
        grid=(B, NB),
        in_specs=[
            pl.BlockSpec((1, BLK, F2), lambda b, nb: (b, nb, 0)),
            pl.BlockSpec((B, NB, 2, F2), lambda b, nb: (0, 0, 0, 0)),
            pl.BlockSpec((1, F2), lambda b, nb: (0, 0)),
            pl.BlockSpec((1, F2), lambda b, nb: (0, 0)),
        ],
        out_specs=pl.BlockSpec((1, F2, BLK), lambda b, nb: (b, 0, nb)),
        out_shape=jax.ShapeDtypeStruct((B, F2, N), jnp.float32),
    )(x2, st2, g2r, be2r)

    return out


# speedup vs baseline: 29.3773x; 1.0104x over previous
import functools

import jax
import jax.numpy as jnp
from jax.experimental import pallas as pl
from jax.experimental.pallas import tpu as pltpu


def _stage1(c1_ref, c2_ref, f1_ref, f2hi_ref, w1a_ref, w1b_ref,
            x1_ref, st_ref, *, blk_n, m):
    c1 = c1_ref[0]
    c2 = c2_ref[0]
    cross = jax.lax.dot_general(
        c1, c2, (((1,), (0,)), ((), ())),
        precision=jax.lax.Precision.DEFAULT,
        preferred_element_type=jnp.float32)
    s1 = c1[:, 0:1] ** 2 + c1[:, 1:2] ** 2 + c1[:, 2:3] ** 2
    s2 = c2[0:1, :] ** 2 + c2[1:2, :] ** 2 + c2[2:3, :] ** 2
    d = (-2.0 * cross + s1) + s2

    big = jnp.float32(1e30)
    recips = []
    masks = []
    counts = []
    for _ in range(3):
        mj = jnp.min(d, axis=1, keepdims=True)
        e = (d == mj)
        ef = e.astype(jnp.float32)
        counts.append(jnp.sum(ef, axis=1, keepdims=True))
        recips.append(1.0 / (mj + 1e-8))
        masks.append(ef)
        d = jnp.where(e, big, d)
    three = jnp.float32(3.0)
    take1 = jnp.minimum(counts[0], three)
    take2 = jnp.minimum(counts[1], three - take1)
    take3 = jnp.minimum(counts[2], three - take1 - take2)
    takes = [take1, take2, take3]
    norm = take1 * recips[0] + take2 * recips[1] + take3 * recips[2]

    wrow = jnp.zeros((blk_n, m), jnp.float32)
    for r, ef, c, t in zip(recips, masks, counts, takes):
        wrow = wrow + ef * (r * t / (c * norm))
    f2 = f2hi_ref[0]
    dims = (((1,), (1,)), ((), ()))
    interp = jax.lax.dot_general(wrow, f2, dims,
                                 preferred_element_type=jnp.float32)

    f1 = f1_ref[0]
    xa = jax.lax.dot_general(
        f1, w1a_ref[...], (((0,), (0,)), ((), ())),
        preferred_element_type=jnp.float32)
    xb = jnp.dot(interp, w1b_ref[...],
                 preferred_element_type=jnp.float32)
    x = xa + xb
    x1_ref[0] = x

    s = jnp.sum(x, axis=0)
    sq = jnp.sum(x * x, axis=0)
    st_ref[0, 0] = jnp.concatenate([s[None, :], sq[None, :]], axis=0)


def _bn_from_partials(st_ref, n_total):
    st = jnp.sum(st_ref[...], axis=(0, 1))
    mean = st[0:1, :] / n_total
    var = st[1:2, :] / n_total - mean * mean
    rstd = jax.lax.rsqrt(var + 1e-5)
    return mean, rstd


def _stage23(x1_ref, st_ref, g1_ref, be1_ref, w2t_ref, g2_ref, be2_ref,
             out_ref, x2_scr, st2_scr, *, n_total):
    ph = pl.program_id(0)
    step = pl.program_id(1) * pl.num_programs(2) + pl.program_id(2)

    @pl.when(ph == 0)
    def _phase0():
        x = x1_ref[0]
        mean, rstd = _bn_from_partials(st_ref, n_total)
        h = (x - mean) * (rstd * g1_ref[...]) + be1_ref[...]
        h = jnp.maximum(h, 0.0)
        y = jnp.dot(h, w2t_ref[...],
                    preferred_element_type=jnp.float32)
        x2_scr[step] = y
        s = jnp.sum(y, axis=0)
        sq = jnp.sum(y * y, axis=0)
        part = jnp.concatenate([s[None, :], sq[None, :]], axis=0)

        @pl.when(step == 0)
        def _():
            st2_scr[...] = part

        @pl.when(step != 0)
        def _():
            st2_scr[...] = st2_scr[...] + part

    @pl.when(ph == 1)
    def _phase1():
        st = st2_scr[...]
        mean = st[0:1, :] / n_total
        var = st[1:2, :] / n_total - mean * mean
        rstd = jax.lax.rsqrt(var + 1e-5)
        h = (x2_scr[step] - mean) * (rstd * g2_ref[...]) + be2_ref[...]
        h = jnp.maximum(h, 0.0)
        out_ref[0] = h.T


def kernel(feature1, coord1, feature2, coord2, W1, b1, g1, be1, W2, b2, g2, be2):
    B, C1, N = feature1.shape
    _, C2, M = feature2.shape
    F1 = W1.shape[0]
    F2 = W2.shape[0]
    BLK = 512
    NB = N // BLK
    n_total = float(B * N)

    c1t = coord1.transpose(0, 2, 1)
    w1a = W1[:, :C1].T
    w1b = W1[:, C1:].T
    w2t = W2.T
    g1r, be1r = g1.reshape(1, F1), be1.reshape(1, F1)
    g2r, be2r = g2.reshape(1, F2), be2.reshape(1, F2)

    x1, st1 = pl.pallas_call(
        functools.partial(_stage1, blk_n=BLK, m=M),
        grid=(B, NB),
        in_specs=[
            pl.BlockSpec((1, BLK, 3), lambda b, nb: (b, nb, 0)),
            pl.BlockSpec((1, 3, M), lambda b, nb: (b, 0, 0)),
            pl.BlockSpec((1, C1, BLK), lambda b, nb: (b, 0, nb)),
            pl.BlockSpec((1, C2, M), lambda b, nb: (b, 0, 0)),
            pl.BlockSpec((C1, F1), lambda b, nb: (0, 0)),
            pl.BlockSpec((C2, F1), lambda b, nb: (0, 0)),
        ],
        out_specs=[
            pl.BlockSpec((1, BLK, F1), lambda b, nb: (b, nb, 0)),
            pl.BlockSpec((1, 1, 2, F1), lambda b, nb: (b, nb, 0, 0)),
        ],
        out_shape=[
            jax.ShapeDtypeStruct((B, N, F1), jnp.float32),
            jax.ShapeDtypeStruct((B, NB, 2, F1), jnp.float32),
        ],
    )(c1t, coord2, feature1, feature2, w1a, w1b)

    out = pl.pallas_call(
        functools.partial(_stage23, n_total=n_total),
        grid=(2, B, NB),
        in_specs=[
            pl.BlockSpec((1, BLK, F1), lambda ph, b, nb: (b, nb, 0)),
            pl.BlockSpec((B, NB, 2, F1), lambda ph, b, nb: (0, 0, 0, 0)),
            pl.BlockSpec((1, F1), lambda ph, b, nb: (0, 0)),
            pl.BlockSpec((1, F1), lambda ph, b, nb: (0, 0)),
            pl.BlockSpec((F1, F2), lambda ph, b, nb: (0, 0)),
            pl.BlockSpec((1, F2), lambda ph, b, nb: (0, 0)),
            pl.BlockSpec((1, F2), lambda ph, b, nb: (0, 0)),
        ],
        out_specs=pl.BlockSpec((1, F2, BLK),
                               lambda ph, b, nb: (b * ph, 0, nb * ph)),
        out_shape=jax.ShapeDtypeStruct((B, F2, N), jnp.float32),
        scratch_shapes=[
            pltpu.VMEM((B * NB, BLK, F2), jnp.float32),
            pltpu.VMEM((2, F2), jnp.float32),
        ],
    )(x1, st1, g1r, be1r, w2t, g2r, be2r)

    return out


# speedup vs baseline: 37.4129x; 1.2735x over previous
import functools

import jax
import jax.numpy as jnp
from jax.experimental import pallas as pl
from jax.experimental.pallas import tpu as pltpu


def _stage1(c1_ref, c2_ref, f1_ref, f2hi_ref, w1a_ref, w1b_ref,
            x1_ref, st_ref, *, blk_n, m):
    c1 = c1_ref[0]
    c2 = c2_ref[0]
    cross = jax.lax.dot_general(
        c1, c2, (((1,), (0,)), ((), ())),
        precision=jax.lax.Precision.DEFAULT,
        preferred_element_type=jnp.float32)
    s1 = c1[:, 0:1] ** 2 + c1[:, 1:2] ** 2 + c1[:, 2:3] ** 2
    s2 = c2[0:1, :] ** 2 + c2[1:2, :] ** 2 + c2[2:3, :] ** 2
    d = (-2.0 * cross + s1) + s2

    big = jnp.float32(1e30)
    recips = []
    masks = []
    counts = []
    for _ in range(3):
        mj = jnp.min(d, axis=1, keepdims=True)
        e = (d == mj)
        ef = e.astype(jnp.float32)
        counts.append(jnp.sum(ef, axis=1, keepdims=True))
        recips.append(1.0 / (mj + 1e-8))
        masks.append(ef)
        d = jnp.where(e, big, d)
    three = jnp.float32(3.0)
    take1 = jnp.minimum(counts[0], three)
    take2 = jnp.minimum(counts[1], three - take1)
    take3 = jnp.minimum(counts[2], three - take1 - take2)
    takes = [take1, take2, take3]
    norm = take1 * recips[0] + take2 * recips[1] + take3 * recips[2]

    wrow = jnp.zeros((blk_n, m), jnp.float32)
    for r, ef, c, t in zip(recips, masks, counts, takes):
        wrow = wrow + ef * (r * t / (c * norm))
    f2 = f2hi_ref[0]
    dims = (((1,), (1,)), ((), ()))
    interp = jax.lax.dot_general(wrow, f2, dims,
                                 preferred_element_type=jnp.float32)

    f1 = f1_ref[0]
    xa = jax.lax.dot_general(
        f1, w1a_ref[...], (((0,), (0,)), ((), ())),
        preferred_element_type=jnp.float32)
    xb = jnp.dot(interp, w1b_ref[...],
                 preferred_element_type=jnp.float32)
    x = xa + xb
    x1_ref[0] = x

    s = jnp.sum(x, axis=0)
    sq = jnp.sum(x * x, axis=0)
    st_ref[0, 0] = jnp.concatenate([s[None, :], sq[None, :]], axis=0)


def _bn_from_partials(st_ref, n_total):
    st = jnp.sum(st_ref[...], axis=(0, 1))
    mean = st[0:1, :] / n_total
    var = st[1:2, :] / n_total - mean * mean
    rstd = jax.lax.rsqrt(var + 1e-5)
    return mean, rstd


def _stage23(x1_ref, st_ref, g1_ref, be1_ref, w2t_ref, g2_ref, be2_ref,
             out_ref, x2_scr, st2_scr, *, n_total):
    ph = pl.program_id(0)
    step = pl.program_id(1) * pl.num_programs(2) + pl.program_id(2)

    @pl.when(ph == 0)
    def _phase0():
        x = x1_ref[0]
        mean, rstd = _bn_from_partials(st_ref, n_total)
        h = (x - mean) * (rstd * g1_ref[...]) + be1_ref[...]
        h = jnp.maximum(h, 0.0)
        y = jnp.dot(h, w2t_ref[...],
                    preferred_element_type=jnp.float32)
        x2_scr[step] = y
        s = jnp.sum(y, axis=0)
        sq = jnp.sum(y * y, axis=0)
        part = jnp.concatenate([s[None, :], sq[None, :]], axis=0)

        @pl.when(step == 0)
        def _():
            st2_scr[...] = part

        @pl.when(step != 0)
        def _():
            st2_scr[...] = st2_scr[...] + part

    @pl.when(ph == 1)
    def _phase1():
        st = st2_scr[...]
        mean = st[0:1, :] / n_total
        var = st[1:2, :] / n_total - mean * mean
        rstd = jax.lax.rsqrt(var + 1e-5)
        h = (x2_scr[step] - mean) * (rstd * g2_ref[...]) + be2_ref[...]
        h = jnp.maximum(h, 0.0)
        out_ref[0] = h.T


def kernel(feature1, coord1, feature2, coord2, W1, b1, g1, be1, W2, b2, g2, be2):
    B, C1, N = feature1.shape
    _, C2, M = feature2.shape
    F1 = W1.shape[0]
    F2 = W2.shape[0]
    BLK = 512
    NB = N // BLK
    n_total = float(B * N)

    c1t = coord1.transpose(0, 2, 1)
    w1a = W1[:, :C1].T
    w1b = W1[:, C1:].T
    w2t = W2.T
    g1r, be1r = g1.reshape(1, F1), be1.reshape(1, F1)
    g2r, be2r = g2.reshape(1, F2), be2.reshape(1, F2)

    x1, st1 = pl.pallas_call(
        functools.partial(_stage1, blk_n=BLK, m=M),
        grid=(B, NB),
        in_specs=[
            pl.BlockSpec((1, BLK, 3), lambda b, nb: (b, nb, 0)),
            pl.BlockSpec((1, 3, M), lambda b, nb: (b, 0, 0)),
            pl.BlockSpec((1, C1, BLK), lambda b, nb: (b, 0, nb)),
            pl.BlockSpec((1, C2, M), lambda b, nb: (b, 0, 0)),
            pl.BlockSpec((C1, F1), lambda b, nb: (0, 0)),
            pl.BlockSpec((C2, F1), lambda b, nb: (0, 0)),
        ],
        out_specs=[
            pl.BlockSpec((1, BLK, F1), lambda b, nb: (b, nb, 0)),
            pl.BlockSpec((1, 1, 2, F1), lambda b, nb: (b, nb, 0, 0)),
        ],
        out_shape=[
            jax.ShapeDtypeStruct((B, N, F1), jnp.float32),
            jax.ShapeDtypeStruct((B, NB, 2, F1), jnp.float32),
        ],
    )(c1t, coord2, feature1, feature2, w1a, w1b)

    BLK2 = 2048
    NB2 = N // BLK2
    out = pl.pallas_call(
        functools.partial(_stage23, n_total=n_total),
        grid=(2, B, NB2),
        in_specs=[
            pl.BlockSpec((1, BLK2, F1),
                         lambda ph, b, nb: (b * (1 - ph), nb * (1 - ph), 0)),
            pl.BlockSpec((B, NB, 2, F1), lambda ph, b, nb: (0, 0, 0, 0)),
            pl.BlockSpec((1, F1), lambda ph, b, nb: (0, 0)),
            pl.BlockSpec((1, F1), lambda ph, b, nb: (0, 0)),
            pl.BlockSpec((F1, F2), lambda ph, b, nb: (0, 0)),
            pl.BlockSpec((1, F2), lambda ph, b, nb: (0, 0)),
            pl.BlockSpec((1, F2), lambda ph, b, nb: (0, 0)),
        ],
        out_specs=pl.BlockSpec((1, F2, BLK2),
                               lambda ph, b, nb: (b * ph, 0, nb * ph)),
        out_shape=jax.ShapeDtypeStruct((B, F2, N), jnp.float32),
        scratch_shapes=[
            pltpu.VMEM((B * NB2, BLK2, F2), jnp.float32),
            pltpu.VMEM((2, F2), jnp.float32),
        ],
    )(x1, st1, g1r, be1r, w2t, g2r, be2r)

    return out


# speedup vs baseline: 38.0259x; 1.0164x over previous
import functools

import jax
import jax.numpy as jnp
from jax.experimental import pallas as pl
from jax.experimental.pallas import tpu as pltpu


def _stage1(c1_ref, c2_ref, f1_ref, f2hi_ref, w1a_ref, w1b_ref,
            x1_ref, st_ref, *, blk_n, m):
    c1 = c1_ref[0]
    c2 = c2_ref[0]
    cross = jax.lax.dot_general(
        c1, c2, (((1,), (0,)), ((), ())),
        precision=jax.lax.Precision.DEFAULT,
        preferred_element_type=jnp.float32)
    s1 = c1[:, 0:1] ** 2 + c1[:, 1:2] ** 2 + c1[:, 2:3] ** 2
    s2 = c2[0:1, :] ** 2 + c2[1:2, :] ** 2 + c2[2:3, :] ** 2
    d = (-2.0 * cross + s1) + s2

    big = jnp.float32(1e30)
    recips = []
    masks = []
    counts = []
    for _ in range(3):
        mj = jnp.min(d, axis=1, keepdims=True)
        e = (d == mj)
        ef = e.astype(jnp.float32)
        counts.append(jnp.sum(ef, axis=1, keepdims=True))
        recips.append(1.0 / (mj + 1e-8))
        masks.append(ef)
        d = jnp.where(e, big, d)
    three = jnp.float32(3.0)
    take1 = jnp.minimum(counts[0], three)
    take2 = jnp.minimum(counts[1], three - take1)
    take3 = jnp.minimum(counts[2], three - take1 - take2)
    takes = [take1, take2, take3]
    norm = take1 * recips[0] + take2 * recips[1] + take3 * recips[2]

    wrow = jnp.zeros((blk_n, m), jnp.float32)
    for r, ef, c, t in zip(recips, masks, counts, takes):
        wrow = wrow + ef * (r * t / (c * norm))
    f2 = f2hi_ref[0]
    dims = (((1,), (1,)), ((), ()))
    interp = jax.lax.dot_general(wrow, f2, dims,
                                 preferred_element_type=jnp.float32)

    f1 = f1_ref[0]
    xa = jax.lax.dot_general(
        f1, w1a_ref[...], (((0,), (0,)), ((), ())),
        preferred_element_type=jnp.float32)
    xb = jnp.dot(interp, w1b_ref[...],
                 preferred_element_type=jnp.float32)
    x = xa + xb
    x1_ref[0] = x.astype(jnp.bfloat16)

    s = jnp.sum(x, axis=0)
    sq = jnp.sum(x * x, axis=0)
    st_ref[0, 0] = jnp.concatenate([s[None, :], sq[None, :]], axis=0)


def _bn_from_partials(st_ref, n_total):
    st = jnp.sum(st_ref[...], axis=(0, 1))
    mean = st[0:1, :] / n_total
    var = st[1:2, :] / n_total - mean * mean
    rstd = jax.lax.rsqrt(var + 1e-5)
    return mean, rstd


def _stage23(x1_ref, st_ref, g1_ref, be1_ref, w2t_ref, g2_ref, be2_ref,
             out_ref, x2_scr, st2_scr, *, n_total):
    ph = pl.program_id(0)
    step = pl.program_id(1) * pl.num_programs(2) + pl.program_id(2)

    @pl.when(ph == 0)
    def _phase0():
        x = x1_ref[0].astype(jnp.float32)
        mean, rstd = _bn_from_partials(st_ref, n_total)
        h = (x - mean) * (rstd * g1_ref[...]) + be1_ref[...]
        h = jnp.maximum(h, 0.0)
        y = jnp.dot(h, w2t_ref[...],
                    preferred_element_type=jnp.float32)
        x2_scr[step] = y
        s = jnp.sum(y, axis=0)
        sq = jnp.sum(y * y, axis=0)
        part = jnp.concatenate([s[None, :], sq[None, :]], axis=0)

        @pl.when(step == 0)
        def _():
            st2_scr[...] = part

        @pl.when(step != 0)
        def _():
            st2_scr[...] = st2_scr[...] + part

    @pl.when(ph == 1)
    def _phase1():
        st = st2_scr[...]
        mean = st[0:1, :] / n_total
        var = st[1:2, :] / n_total - mean * mean
        rstd = jax.lax.rsqrt(var + 1e-5)
        h = (x2_scr[step] - mean) * (rstd * g2_ref[...]) + be2_ref[...]
        h = jnp.maximum(h, 0.0)
        out_ref[0] = h.T


def kernel(feature1, coord1, feature2, coord2, W1, b1, g1, be1, W2, b2, g2, be2):
    B, C1, N = feature1.shape
    _, C2, M = feature2.shape
    F1 = W1.shape[0]
    F2 = W2.shape[0]
    BLK = 512
    NB = N // BLK
    n_total = float(B * N)

    c1t = coord1.transpose(0, 2, 1)
    w1a = W1[:, :C1].T
    w1b = W1[:, C1:].T
    w2t = W2.T
    g1r, be1r = g1.reshape(1, F1), be1.reshape(1, F1)
    g2r, be2r = g2.reshape(1, F2), be2.reshape(1, F2)

    x1, st1 = pl.pallas_call(
        functools.partial(_stage1, blk_n=BLK, m=M),
        grid=(B, NB),
        in_specs=[
            pl.BlockSpec((1, BLK, 3), lambda b, nb: (b, nb, 0)),
            pl.BlockSpec((1, 3, M), lambda b, nb: (b, 0, 0)),
            pl.BlockSpec((1, C1, BLK), lambda b, nb: (b, 0, nb)),
            pl.BlockSpec((1, C2, M), lambda b, nb: (b, 0, 0)),
            pl.BlockSpec((C1, F1), lambda b, nb: (0, 0)),
            pl.BlockSpec((C2, F1), lambda b, nb: (0, 0)),
        ],
        out_specs=[
            pl.BlockSpec((1, BLK, F1), lambda b, nb: (b, nb, 0)),
            pl.BlockSpec((1, 1, 2, F1), lambda b, nb: (b, nb, 0, 0)),
        ],
        out_shape=[
            jax.ShapeDtypeStruct((B, N, F1), jnp.bfloat16),
            jax.ShapeDtypeStruct((B, NB, 2, F1), jnp.float32),
        ],
    )(c1t, coord2, feature1, feature2, w1a, w1b)

    BLK2 = 2048
    NB2 = N // BLK2
    out = pl.pallas_call(
        functools.partial(_stage23, n_total=n_total),
        grid=(2, B, NB2),
        in_specs=[
            pl.BlockSpec((1, BLK2, F1),
                         lambda ph, b, nb: (b * (1 - ph), nb * (1 - ph), 0)),
            pl.BlockSpec((B, NB, 2, F1), lambda ph, b, nb: (0, 0, 0, 0)),
            pl.BlockSpec((1, F1), lambda ph, b, nb: (0, 0)),
            pl.BlockSpec((1, F1), lambda ph, b, nb: (0, 0)),
            pl.BlockSpec((F1, F2), lambda ph, b, nb: (0, 0)),
            pl.BlockSpec((1, F2), lambda ph, b, nb: (0, 0)),
            pl.BlockSpec((1, F2), lambda ph, b, nb: (0, 0)),
        ],
        out_specs=pl.BlockSpec((1, F2, BLK2),
                               lambda ph, b, nb: (b * ph, 0, nb * ph)),
        out_shape=jax.ShapeDtypeStruct((B, F2, N), jnp.float32),
        scratch_shapes=[
            pltpu.VMEM((B * NB2, BLK2, F2), jnp.float32),
            pltpu.VMEM((2, F2), jnp.float32),
        ],
    )(x1, st1, g1r, be1r, w2t, g2r, be2r)

    return out
